# Initial kernel scaffold; baseline (speedup 1.0000x reference)
import jax
import jax.numpy as jnp
from jax.experimental import pallas as pl


def kernel(x, edge_index, eps, gamma, beta, W1, b1, W2, b2, Wmu, bmu, Wvar, bvar, Wd1, bd1, Wd2, bd2, Wd3, bd3):
    raise NotImplementedError("write your pallas kernel here")


# speedup vs baseline: 6.2993x; 6.2993x over previous
import functools

import jax
import jax.numpy as jnp
from jax import lax
from jax.experimental import pallas as pl
from jax.experimental.pallas import tpu as pltpu
from jax.experimental.pallas import tpu_sc as plsc

N = 10000
E = 320000
D = 128
BIG = 32
HLAT = 2

NC = 2
NS = 16
NW = NC * NS
RG = 128
NROWS = E // RG
ROWS_PER_W = NROWS // NW
EXTRA = NROWS - ROWS_PER_W * NW
NPT = N // NS

_mesh = plsc.VectorSubcoreMesh(
    core_axis_name="c", subcore_axis_name="s", num_cores=NC, num_subcores=NS
)

_sc_params = pltpu.CompilerParams(use_tc_tiling_on_sc=False)


def _worker_rows(wid):
    base = wid * ROWS_PER_W + jnp.minimum(wid, EXTRA)
    nrows = jnp.where(wid < EXTRA, ROWS_PER_W + 1, ROWS_PER_W)
    return base, nrows


@functools.partial(
    pl.kernel,
    out_type=jax.ShapeDtypeStruct((NROWS, RG, BIG), jnp.float32),
    mesh=_mesh,
    compiler_params=_sc_params,
    scratch_types=[
        pltpu.VMEM((RG,), jnp.int32),
        pltpu.VMEM((RG,), jnp.int32),
        pltpu.VMEM((RG, BIG), jnp.float32),
        pltpu.VMEM((RG, BIG), jnp.float32),
        pltpu.SemaphoreType.DMA,
        pltpu.SemaphoreType.DMA,
    ],
)
def _sc_gather(p_hbm, q_hbm, dst_hbm, src_hbm, h_hbm, ds_v, sr_v, a_v, b_v, sem1, sem2):
    wid = lax.axis_index("s") * NC + lax.axis_index("c")
    base, nrows = _worker_rows(wid)

    def body(r, carry):
        row = base + r
        pltpu.sync_copy(dst_hbm.at[row], ds_v)
        pltpu.sync_copy(src_hbm.at[row], sr_v)
        c1 = pltpu.async_copy(p_hbm.at[ds_v], a_v, sem1)
        c2 = pltpu.async_copy(q_hbm.at[sr_v], b_v, sem2)
        c1.wait()
        c2.wait()

        def add_row(i, c):
            a_v[i, 0:16] = a_v[i, 0:16] + b_v[i, 0:16]
            a_v[i, 16:32] = a_v[i, 16:32] + b_v[i, 16:32]
            return c

        lax.fori_loop(0, RG, add_row, 0)
        pltpu.sync_copy(a_v, h_hbm.at[row])
        return carry

    lax.fori_loop(0, nrows, body, 0)


@functools.partial(
    pl.kernel,
    out_type=(
        jax.ShapeDtypeStruct((NC, N, BIG), jnp.float32),
        jax.ShapeDtypeStruct((NC, N, 16), jnp.float32),
    ),
    mesh=_mesh,
    compiler_params=_sc_params,
    scratch_types=[
        pltpu.VMEM((RG,), jnp.int32),
        pltpu.VMEM((RG, BIG), jnp.float32),
        pltpu.VMEM((RG, 16), jnp.float32),
        pltpu.VMEM_SHARED((N, BIG), jnp.float32),
        pltpu.VMEM_SHARED((N, 16), jnp.float32),
    ],
)
def _sc_scatter_cnt(m_hbm, dst_hbm, z32_hbm, z16_hbm, ones_hbm, sm_hbm, sc_hbm,
                    ds_v, m_v, ones_v, accm, accc):
    cid = lax.axis_index("c")
    sid = lax.axis_index("s")
    wid = sid * NC + cid
    t0 = sid * NPT
    pltpu.sync_copy(z32_hbm.at[pl.ds(t0, NPT)], accm.at[pl.ds(t0, NPT)])
    pltpu.sync_copy(z16_hbm.at[pl.ds(t0, NPT)], accc.at[pl.ds(t0, NPT)])
    pltpu.sync_copy(ones_hbm, ones_v)
    plsc.subcore_barrier()

    base, nrows = _worker_rows(wid)

    def body(r, carry):
        row = base + r
        pltpu.sync_copy(dst_hbm.at[row], ds_v)
        pltpu.sync_copy(m_hbm.at[row], m_v)
        pltpu.sync_copy(m_v, accm.at[ds_v], add=True)
        pltpu.sync_copy(ones_v, accc.at[ds_v], add=True)
        return carry

    lax.fori_loop(0, nrows, body, 0)
    plsc.subcore_barrier()
    pltpu.sync_copy(accm.at[pl.ds(t0, NPT)], sm_hbm.at[cid, pl.ds(t0, NPT)])
    pltpu.sync_copy(accc.at[pl.ds(t0, NPT)], sc_hbm.at[cid, pl.ds(t0, NPT)])


@functools.partial(
    pl.kernel,
    out_type=jax.ShapeDtypeStruct((NC, N, BIG), jnp.float32),
    mesh=_mesh,
    compiler_params=_sc_params,
    scratch_types=[
        pltpu.VMEM((RG,), jnp.int32),
        pltpu.VMEM((RG, BIG), jnp.float32),
        pltpu.VMEM_SHARED((N, BIG), jnp.float32),
    ],
)
def _sc_scatter(m_hbm, dst_hbm, z32_hbm, sm_hbm, ds_v, m_v, accm):
    cid = lax.axis_index("c")
    sid = lax.axis_index("s")
    wid = sid * NC + cid
    t0 = sid * NPT
    pltpu.sync_copy(z32_hbm.at[pl.ds(t0, NPT)], accm.at[pl.ds(t0, NPT)])
    plsc.subcore_barrier()

    base, nrows = _worker_rows(wid)

    def body(r, carry):
        row = base + r
        pltpu.sync_copy(dst_hbm.at[row], ds_v)
        pltpu.sync_copy(m_hbm.at[row], m_v)
        pltpu.sync_copy(m_v, accm.at[ds_v], add=True)
        return carry

    lax.fori_loop(0, nrows, body, 0)
    plsc.subcore_barrier()
    pltpu.sync_copy(accm.at[pl.ds(t0, NPT)], sm_hbm.at[cid, pl.ds(t0, NPT)])


def _tc_prep_body(x_ref, g_ref, bt_ref, w1d_ref, w1b_ref, b1_ref, p_ref, q_ref):
    x = x_ref[...]
    mean = jnp.mean(x, axis=0, keepdims=True)
    xc = x - mean
    var = jnp.mean(xc * xc, axis=0, keepdims=True)
    xn = xc * lax.rsqrt(var + 1e-5) * g_ref[...] + bt_ref[...]
    p_ref[...] = (
        jnp.dot(xn, w1d_ref[...], preferred_element_type=jnp.float32) + b1_ref[...]
    )
    q_ref[...] = jnp.dot(xn, w1b_ref[...], preferred_element_type=jnp.float32)


_tc_prep = pl.pallas_call(
    _tc_prep_body,
    out_shape=(
        jax.ShapeDtypeStruct((N, BIG), jnp.float32),
        jax.ShapeDtypeStruct((N, BIG), jnp.float32),
    ),
)


EPACK = E // 4
BE = 8000


def _tc_mlp_body(h_ref, w_ref, b_ref, m_ref):
    h = jnp.maximum(h_ref[...], 0.0)
    m = jnp.dot(h, w_ref[...], preferred_element_type=jnp.float32) + b_ref[...]
    m_ref[...] = jnp.maximum(m, 0.0)


_tc_mlp = pl.pallas_call(
    _tc_mlp_body,
    grid=(EPACK // BE,),
    in_specs=[
        pl.BlockSpec((BE, D), lambda i: (i, 0)),
        pl.BlockSpec((D, D), lambda i: (0, 0)),
        pl.BlockSpec((1, D), lambda i: (0, 0)),
    ],
    out_specs=pl.BlockSpec((BE, D), lambda i: (i, 0)),
    out_shape=jax.ShapeDtypeStruct((EPACK, D), jnp.float32),
)


def _tc_mid_body(sm_ref, sc_ref, eps_ref, wmu_ref, bmu_ref, wvar_ref, bvar_ref,
                 wd1d_ref, wd1b_ref, bd1_ref,
                 mu_ref, lv_ref, r_ref, s_ref, cnt_ref):
    s = sm_ref[0] + sm_ref[1]
    cnt = sc_ref[0, :, 0:1] + sc_ref[1, :, 0:1]
    henc = s / jnp.maximum(cnt, 1.0)
    mu = jnp.dot(henc, wmu_ref[...], preferred_element_type=jnp.float32) + bmu_ref[...]
    lv = jnp.dot(henc, wvar_ref[...], preferred_element_type=jnp.float32) + bvar_ref[...]
    z = mu + eps_ref[...] * jnp.exp(0.5 * lv)
    z0 = z[:, 0:1]
    z1 = z[:, 1:2]
    r_ref[...] = z0 * wd1d_ref[0:1, :] + z1 * wd1d_ref[1:2, :] + bd1_ref[...]
    s_ref[...] = z0 * wd1b_ref[0:1, :] + z1 * wd1b_ref[1:2, :]
    mu_ref[...] = mu
    lv_ref[...] = lv
    cnt_ref[...] = cnt


_tc_mid = pl.pallas_call(
    _tc_mid_body,
    out_shape=(
        jax.ShapeDtypeStruct((N, HLAT), jnp.float32),
        jax.ShapeDtypeStruct((N, HLAT), jnp.float32),
        jax.ShapeDtypeStruct((N, BIG), jnp.float32),
        jax.ShapeDtypeStruct((N, BIG), jnp.float32),
        jax.ShapeDtypeStruct((N, 1), jnp.float32),
    ),
)


def _tc_out_body(tm_ref, cnt_ref, wd3_ref, bd3_ref, out_ref):
    t = tm_ref[0] + tm_ref[1]
    cnt = cnt_ref[...]
    tmean = t / jnp.maximum(cnt, 1.0)
    mask = jnp.where(cnt > 0, 1.0, 0.0)
    out_ref[...] = (
        jnp.dot(tmean, wd3_ref[...], preferred_element_type=jnp.float32)
        + bd3_ref[...] * mask
    )


_tc_out = pl.pallas_call(
    _tc_out_body,
    out_shape=jax.ShapeDtypeStruct((N, D), jnp.float32),
)


def kernel(x, edge_index, eps, gamma, beta, W1, b1, W2, b2, Wmu, bmu, Wvar, bvar,
           Wd1, bd1, Wd2, bd2, Wd3, bd3):
    src = edge_index[0].reshape(NROWS, RG)
    dst = edge_index[1].reshape(NROWS, RG)
    w1a, w1b = W1[:D], W1[D:]

    p_tab, q_tab = _tc_prep(
        x, gamma.reshape(1, D), beta.reshape(1, D), w1a - w1b, w1b, b1.reshape(1, BIG)
    )

    eye4 = jnp.eye(4, dtype=jnp.float32)
    w2blk = jnp.kron(eye4, W2)
    b2t = jnp.tile(b2, 4).reshape(1, D)

    h1 = _sc_gather(p_tab, q_tab, dst, src)
    m1 = _tc_mlp(h1.reshape(EPACK, D), w2blk, b2t)

    z32 = jnp.zeros((N, BIG), jnp.float32)
    z16 = jnp.zeros((N, 16), jnp.float32)
    ones16 = jnp.ones((RG, 16), jnp.float32)
    sm, sc = _sc_scatter_cnt(m1.reshape(NROWS, RG, BIG), dst, z32, z16, ones16)

    wd1a, wd1b = Wd1[:HLAT], Wd1[HLAT:]
    mu, lv, r_tab, s_tab, cnt = _tc_mid(
        sm, sc, eps, Wmu, bmu.reshape(1, HLAT), Wvar, bvar.reshape(1, HLAT),
        wd1a - wd1b, wd1b, bd1.reshape(1, BIG)
    )

    wd2blk = jnp.kron(eye4, Wd2)
    bd2t = jnp.tile(bd2, 4).reshape(1, D)
    h2 = _sc_gather(r_tab, s_tab, dst, src)
    m2 = _tc_mlp(h2.reshape(EPACK, D), wd2blk, bd2t)
    tm = _sc_scatter(m2.reshape(NROWS, RG, BIG), dst, z32)

    out = _tc_out(tm, cnt, Wd3, bd3.reshape(1, D))
    return (out, mu, lv)


# speedup vs baseline: 11.8650x; 1.8835x over previous
import functools

import jax
import jax.numpy as jnp
from jax import lax
from jax.experimental import pallas as pl
from jax.experimental.pallas import tpu as pltpu
from jax.experimental.pallas import tpu_sc as plsc

N = 10000
E = 320000
D = 128
BIG = 32
HLAT = 2

NC = 2
NS = 16
NW = NC * NS
RG = 125
NROWS = E // RG
RPW = NROWS // NW
NPT = N // NS

_mesh = plsc.VectorSubcoreMesh(
    core_axis_name="c", subcore_axis_name="s", num_cores=NC, num_subcores=NS
)

_sc_params = pltpu.CompilerParams(use_tc_tiling_on_sc=False)


@functools.partial(
    pl.kernel,
    out_type=jax.ShapeDtypeStruct((NROWS, RG, BIG), jnp.float32),
    mesh=_mesh,
    compiler_params=_sc_params,
    scratch_types=[
        pltpu.VMEM((RPW, RG), jnp.int32),
        pltpu.VMEM((RPW, RG), jnp.int32),
        pltpu.VMEM((RG, BIG), jnp.float32),
        pltpu.VMEM((RG, BIG), jnp.float32),
        pltpu.VMEM((RG, BIG), jnp.float32),
        pltpu.VMEM((RG, BIG), jnp.float32),
        pltpu.SemaphoreType.DMA,
        pltpu.SemaphoreType.DMA,
        pltpu.SemaphoreType.DMA,
        pltpu.SemaphoreType.DMA,
    ],
)
def _sc_gather(p_hbm, q_hbm, dst_hbm, src_hbm, h_hbm,
               ds_all, sr_all, a0, b0, a1, b1, g0, g1, w0, w1):
    wid = lax.axis_index("s") * NC + lax.axis_index("c")
    base = wid * RPW
    pltpu.sync_copy(dst_hbm.at[pl.ds(base, RPW)], ds_all)
    pltpu.sync_copy(src_hbm.at[pl.ds(base, RPW)], sr_all)

    def fire(row, av, bv, sem):
        pltpu.async_copy(p_hbm.at[ds_all.at[row]], av, sem)
        pltpu.async_copy(q_hbm.at[sr_all.at[row]], bv, sem)

    def drain_gather(av, sem):
        pltpu.make_async_copy(h_hbm.at[0], av, sem).wait()
        pltpu.make_async_copy(h_hbm.at[0], av, sem).wait()

    def drain_write(av, sem):
        pltpu.make_async_copy(av, h_hbm.at[0], sem).wait()

    def add_rows(av, bv):
        def add_row(i, c):
            av[i, 0:16] = av[i, 0:16] + bv[i, 0:16]
            av[i, 16:32] = av[i, 16:32] + bv[i, 16:32]
            return c
        lax.fori_loop(0, RG, add_row, 0)

    fire(0, a0, b0, g0)

    def body(t, carry):
        r0 = 2 * t

        @pl.when(t > 0)
        def _():
            drain_write(a1, w1)

        fire(r0 + 1, a1, b1, g1)
        drain_gather(a0, g0)
        add_rows(a0, b0)
        pltpu.async_copy(a0, h_hbm.at[base + r0], w0)
        drain_gather(a1, g1)

        @pl.when(t < RPW // 2 - 1)
        def _():
            drain_write(a0, w0)
            fire(r0 + 2, a0, b0, g0)

        add_rows(a1, b1)
        pltpu.async_copy(a1, h_hbm.at[base + r0 + 1], w1)
        return carry

    lax.fori_loop(0, RPW // 2, body, 0)
    drain_write(a0, w0)
    drain_write(a1, w1)


def _scatter_body(m_hbm, dst_hbm, accm, accc, ds_all, m0, m1, ones_v, s0, s1, base):
    def drain(mv, sem):
        pltpu.make_async_copy(m_hbm.at[0], mv, sem).wait()

    pltpu.async_copy(m_hbm.at[base], m0, s0)

    def body(t, carry):
        r0 = 2 * t
        pltpu.async_copy(m_hbm.at[base + r0 + 1], m1, s1)
        drain(m0, s0)
        pltpu.sync_copy(m0, accm.at[ds_all.at[r0]], add=True)
        if accc is not None:
            pltpu.sync_copy(ones_v, accc.at[ds_all.at[r0]], add=True)

        @pl.when(t < RPW // 2 - 1)
        def _():
            pltpu.async_copy(m_hbm.at[base + r0 + 2], m0, s0)

        drain(m1, s1)
        pltpu.sync_copy(m1, accm.at[ds_all.at[r0 + 1]], add=True)
        if accc is not None:
            pltpu.sync_copy(ones_v, accc.at[ds_all.at[r0 + 1]], add=True)
        return carry

    lax.fori_loop(0, RPW // 2, body, 0)


@functools.partial(
    pl.kernel,
    out_type=(
        jax.ShapeDtypeStruct((NC, N, BIG), jnp.float32),
        jax.ShapeDtypeStruct((NC, N, 16), jnp.float32),
    ),
    mesh=_mesh,
    compiler_params=_sc_params,
    scratch_types=[
        pltpu.VMEM((RPW, RG), jnp.int32),
        pltpu.VMEM((RG, BIG), jnp.float32),
        pltpu.VMEM((RG, BIG), jnp.float32),
        pltpu.VMEM((RG, 16), jnp.float32),
        pltpu.VMEM_SHARED((N, BIG), jnp.float32),
        pltpu.VMEM_SHARED((N, 16), jnp.float32),
        pltpu.SemaphoreType.DMA,
        pltpu.SemaphoreType.DMA,
    ],
)
def _sc_scatter_cnt(m_hbm, dst_hbm, z32_hbm, z16_hbm, ones_hbm, sm_hbm, sc_hbm,
                    ds_all, m0, m1, ones_v, accm, accc, s0, s1):
    cid = lax.axis_index("c")
    sid = lax.axis_index("s")
    wid = sid * NC + cid
    base = wid * RPW
    t0 = sid * NPT
    pltpu.sync_copy(dst_hbm.at[pl.ds(base, RPW)], ds_all)
    pltpu.sync_copy(z32_hbm.at[pl.ds(t0, NPT)], accm.at[pl.ds(t0, NPT)])
    pltpu.sync_copy(z16_hbm.at[pl.ds(t0, NPT)], accc.at[pl.ds(t0, NPT)])
    pltpu.sync_copy(ones_hbm, ones_v)
    plsc.subcore_barrier()

    _scatter_body(m_hbm, dst_hbm, accm, accc, ds_all, m0, m1, ones_v, s0, s1, base)

    plsc.subcore_barrier()
    pltpu.sync_copy(accm.at[pl.ds(t0, NPT)], sm_hbm.at[cid, pl.ds(t0, NPT)])
    pltpu.sync_copy(accc.at[pl.ds(t0, NPT)], sc_hbm.at[cid, pl.ds(t0, NPT)])


@functools.partial(
    pl.kernel,
    out_type=jax.ShapeDtypeStruct((NC, N, BIG), jnp.float32),
    mesh=_mesh,
    compiler_params=_sc_params,
    scratch_types=[
        pltpu.VMEM((RPW, RG), jnp.int32),
        pltpu.VMEM((RG, BIG), jnp.float32),
        pltpu.VMEM((RG, BIG), jnp.float32),
        pltpu.VMEM_SHARED((N, BIG), jnp.float32),
        pltpu.SemaphoreType.DMA,
        pltpu.SemaphoreType.DMA,
    ],
)
def _sc_scatter(m_hbm, dst_hbm, z32_hbm, sm_hbm, ds_all, m0, m1, accm, s0, s1):
    cid = lax.axis_index("c")
    sid = lax.axis_index("s")
    wid = sid * NC + cid
    base = wid * RPW
    t0 = sid * NPT
    pltpu.sync_copy(dst_hbm.at[pl.ds(base, RPW)], ds_all)
    pltpu.sync_copy(z32_hbm.at[pl.ds(t0, NPT)], accm.at[pl.ds(t0, NPT)])
    plsc.subcore_barrier()

    _scatter_body(m_hbm, dst_hbm, accm, None, ds_all, m0, m1, None, s0, s1, base)

    plsc.subcore_barrier()
    pltpu.sync_copy(accm.at[pl.ds(t0, NPT)], sm_hbm.at[cid, pl.ds(t0, NPT)])


def _tc_prep_body(x_ref, g_ref, bt_ref, w1d_ref, w1b_ref, b1_ref, p_ref, q_ref):
    x = x_ref[...]
    mean = jnp.mean(x, axis=0, keepdims=True)
    xc = x - mean
    var = jnp.mean(xc * xc, axis=0, keepdims=True)
    xn = xc * lax.rsqrt(var + 1e-5) * g_ref[...] + bt_ref[...]
    p_ref[...] = (
        jnp.dot(xn, w1d_ref[...], preferred_element_type=jnp.float32) + b1_ref[...]
    )
    q_ref[...] = jnp.dot(xn, w1b_ref[...], preferred_element_type=jnp.float32)


_tc_prep = pl.pallas_call(
    _tc_prep_body,
    out_shape=(
        jax.ShapeDtypeStruct((N, BIG), jnp.float32),
        jax.ShapeDtypeStruct((N, BIG), jnp.float32),
    ),
)


EPACK = E // 4
BE = 8000


def _tc_mlp_body(h_ref, w_ref, b_ref, m_ref):
    h = jnp.maximum(h_ref[...], 0.0)
    m = jnp.dot(h, w_ref[...], preferred_element_type=jnp.float32) + b_ref[...]
    m_ref[...] = jnp.maximum(m, 0.0)


_tc_mlp = pl.pallas_call(
    _tc_mlp_body,
    grid=(EPACK // BE,),
    in_specs=[
        pl.BlockSpec((BE, D), lambda i: (i, 0)),
        pl.BlockSpec((D, D), lambda i: (0, 0)),
        pl.BlockSpec((1, D), lambda i: (0, 0)),
    ],
    out_specs=pl.BlockSpec((BE, D), lambda i: (i, 0)),
    out_shape=jax.ShapeDtypeStruct((EPACK, D), jnp.float32),
)


def _tc_mid_body(sm_ref, sc_ref, eps_ref, wmu_ref, bmu_ref, wvar_ref, bvar_ref,
                 wd1d_ref, wd1b_ref, bd1_ref,
                 mu_ref, lv_ref, r_ref, s_ref, cnt_ref):
    s = sm_ref[0] + sm_ref[1]
    cnt = sc_ref[0, :, 0:1] + sc_ref[1, :, 0:1]
    henc = s / jnp.maximum(cnt, 1.0)
    mu = jnp.dot(henc, wmu_ref[...], preferred_element_type=jnp.float32) + bmu_ref[...]
    lv = jnp.dot(henc, wvar_ref[...], preferred_element_type=jnp.float32) + bvar_ref[...]
    z = mu + eps_ref[...] * jnp.exp(0.5 * lv)
    z0 = z[:, 0:1]
    z1 = z[:, 1:2]
    r_ref[...] = z0 * wd1d_ref[0:1, :] + z1 * wd1d_ref[1:2, :] + bd1_ref[...]
    s_ref[...] = z0 * wd1b_ref[0:1, :] + z1 * wd1b_ref[1:2, :]
    mu_ref[...] = mu
    lv_ref[...] = lv
    cnt_ref[...] = cnt


_tc_mid = pl.pallas_call(
    _tc_mid_body,
    out_shape=(
        jax.ShapeDtypeStruct((N, HLAT), jnp.float32),
        jax.ShapeDtypeStruct((N, HLAT), jnp.float32),
        jax.ShapeDtypeStruct((N, BIG), jnp.float32),
        jax.ShapeDtypeStruct((N, BIG), jnp.float32),
        jax.ShapeDtypeStruct((N, 1), jnp.float32),
    ),
)


def _tc_out_body(tm_ref, cnt_ref, wd3_ref, bd3_ref, out_ref):
    t = tm_ref[0] + tm_ref[1]
    cnt = cnt_ref[...]
    tmean = t / jnp.maximum(cnt, 1.0)
    mask = jnp.where(cnt > 0, 1.0, 0.0)
    out_ref[...] = (
        jnp.dot(tmean, wd3_ref[...], preferred_element_type=jnp.float32)
        + bd3_ref[...] * mask
    )


_tc_out = pl.pallas_call(
    _tc_out_body,
    out_shape=jax.ShapeDtypeStruct((N, D), jnp.float32),
)


def kernel(x, edge_index, eps, gamma, beta, W1, b1, W2, b2, Wmu, bmu, Wvar, bvar,
           Wd1, bd1, Wd2, bd2, Wd3, bd3):
    src = edge_index[0].reshape(NROWS, RG)
    dst = edge_index[1].reshape(NROWS, RG)
    w1a, w1b = W1[:D], W1[D:]

    p_tab, q_tab = _tc_prep(
        x, gamma.reshape(1, D), beta.reshape(1, D), w1a - w1b, w1b, b1.reshape(1, BIG)
    )

    eye4 = jnp.eye(4, dtype=jnp.float32)
    w2blk = jnp.kron(eye4, W2)
    b2t = jnp.tile(b2, 4).reshape(1, D)

    h1 = _sc_gather(p_tab, q_tab, dst, src)
    m1 = _tc_mlp(h1.reshape(EPACK, D), w2blk, b2t)

    z32 = jnp.zeros((N, BIG), jnp.float32)
    z16 = jnp.zeros((N, 16), jnp.float32)
    ones16 = jnp.ones((RG, 16), jnp.float32)
    sm, sc = _sc_scatter_cnt(m1.reshape(NROWS, RG, BIG), dst, z32, z16, ones16)

    wd1a, wd1b = Wd1[:HLAT], Wd1[HLAT:]
    mu, lv, r_tab, s_tab, cnt = _tc_mid(
        sm, sc, eps, Wmu, bmu.reshape(1, HLAT), Wvar, bvar.reshape(1, HLAT),
        wd1a - wd1b, wd1b, bd1.reshape(1, BIG)
    )

    wd2blk = jnp.kron(eye4, Wd2)
    bd2t = jnp.tile(bd2, 4).reshape(1, D)
    h2 = _sc_gather(r_tab, s_tab, dst, src)
    m2 = _tc_mlp(h2.reshape(EPACK, D), wd2blk, bd2t)
    tm = _sc_scatter(m2.reshape(NROWS, RG, BIG), dst, z32)

    out = _tc_out(tm, cnt, Wd3, bd3.reshape(1, D))
    return (out, mu, lv)


# speedup vs baseline: 12.2675x; 1.0339x over previous
import functools

import jax
import jax.numpy as jnp
from jax import lax
from jax.experimental import pallas as pl
from jax.experimental.pallas import tpu as pltpu
from jax.experimental.pallas import tpu_sc as plsc

N = 10000
E = 320000
D = 128
BIG = 32
HLAT = 2

NC = 2
NS = 16
NW = NC * NS
RG = 125
NROWS = E // RG
RPW = NROWS // NW
NPT = N // NS

_mesh = plsc.VectorSubcoreMesh(
    core_axis_name="c", subcore_axis_name="s", num_cores=NC, num_subcores=NS
)

_sc_params = pltpu.CompilerParams(use_tc_tiling_on_sc=False)


NBUF = 8
GROUP = RPW // NBUF

_gather_scratch = (
    [pltpu.VMEM((RPW, RG), jnp.int32)] * 2
    + [pltpu.VMEM((RG, BIG), jnp.float32)] * (2 * NBUF)
    + [pltpu.SemaphoreType.DMA] * (2 * NBUF)
)


@functools.partial(
    pl.kernel,
    out_type=jax.ShapeDtypeStruct((NROWS, RG, BIG), jnp.float32),
    mesh=_mesh,
    compiler_params=_sc_params,
    scratch_types=_gather_scratch,
)
def _sc_gather(p_hbm, q_hbm, dst_hbm, src_hbm, h_hbm, ds_all, sr_all, *bufs):
    av = bufs[0:NBUF]
    bv = bufs[NBUF:2 * NBUF]
    gsem = bufs[2 * NBUF:3 * NBUF]
    wsem = bufs[3 * NBUF:4 * NBUF]
    wid = lax.axis_index("s") * NC + lax.axis_index("c")
    base = wid * RPW
    pltpu.sync_copy(dst_hbm.at[pl.ds(base, RPW)], ds_all)
    pltpu.sync_copy(src_hbm.at[pl.ds(base, RPW)], sr_all)

    def fire(row, i):
        pltpu.async_copy(p_hbm.at[ds_all.at[row]], av[i], gsem[i])
        pltpu.async_copy(q_hbm.at[sr_all.at[row]], bv[i], gsem[i])

    def drain_gather(i):
        pltpu.make_async_copy(h_hbm.at[0], av[i], gsem[i]).wait()
        pltpu.make_async_copy(h_hbm.at[0], av[i], gsem[i]).wait()

    def drain_write(i):
        pltpu.make_async_copy(av[i], h_hbm.at[0], wsem[i]).wait()

    def add_rows(i):
        a, b = av[i], bv[i]

        def add_row(k, c):
            a[k, 0:16] = a[k, 0:16] + b[k, 0:16]
            a[k, 16:32] = a[k, 16:32] + b[k, 16:32]
            return c

        lax.fori_loop(0, RG, add_row, 0)

    for i in range(NBUF):
        fire(i, i)

    def body(u, carry):
        r0 = NBUF * u
        for i in range(NBUF):
            drain_gather(i)
            add_rows(i)
            pltpu.async_copy(av[i], h_hbm.at[base + r0 + i], wsem[i])

        @pl.when(u < GROUP - 1)
        def _():
            for i in range(NBUF):
                drain_write(i)
                fire(r0 + NBUF + i, i)

        return carry

    lax.fori_loop(0, GROUP, body, 0)
    for i in range(NBUF):
        drain_write(i)


def _scatter_body(m_hbm, dst_hbm, accm, accc, ds_all, m0, m1, ones_v,
                  s0, s1, sa0, sa1, base):
    def drain_load(mv, sem):
        pltpu.make_async_copy(m_hbm.at[0], mv, sem).wait()

    def drain_add(mv, sem):
        pltpu.make_async_copy(mv, accm.at[ds_all.at[0]], sem).wait()

    pltpu.async_copy(m_hbm.at[base], m0, s0)

    def body(t, carry):
        r0 = 2 * t

        @pl.when(t > 0)
        def _():
            drain_add(m1, sa1)

        pltpu.async_copy(m_hbm.at[base + r0 + 1], m1, s1)
        drain_load(m0, s0)
        pltpu.async_copy(m0, accm.at[ds_all.at[r0]], sa0, add=True)
        if accc is not None:
            pltpu.sync_copy(ones_v, accc.at[ds_all.at[r0]], add=True)

        @pl.when(t < RPW // 2 - 1)
        def _():
            drain_add(m0, sa0)
            pltpu.async_copy(m_hbm.at[base + r0 + 2], m0, s0)

        drain_load(m1, s1)
        pltpu.async_copy(m1, accm.at[ds_all.at[r0 + 1]], sa1, add=True)
        if accc is not None:
            pltpu.sync_copy(ones_v, accc.at[ds_all.at[r0 + 1]], add=True)
        return carry

    lax.fori_loop(0, RPW // 2, body, 0)
    drain_add(m0, sa0)
    drain_add(m1, sa1)


@functools.partial(
    pl.kernel,
    out_type=(
        jax.ShapeDtypeStruct((NC, N, BIG), jnp.float32),
        jax.ShapeDtypeStruct((NC, N, 16), jnp.float32),
    ),
    mesh=_mesh,
    compiler_params=_sc_params,
    scratch_types=[
        pltpu.VMEM((RPW, RG), jnp.int32),
        pltpu.VMEM((RG, BIG), jnp.float32),
        pltpu.VMEM((RG, BIG), jnp.float32),
        pltpu.VMEM((RG, 16), jnp.float32),
        pltpu.VMEM_SHARED((N, BIG), jnp.float32),
        pltpu.VMEM_SHARED((N, 16), jnp.float32),
        pltpu.SemaphoreType.DMA,
        pltpu.SemaphoreType.DMA,
        pltpu.SemaphoreType.DMA,
        pltpu.SemaphoreType.DMA,
    ],
)
def _sc_scatter_cnt(m_hbm, dst_hbm, z32_hbm, z16_hbm, ones_hbm, sm_hbm, sc_hbm,
                    ds_all, m0, m1, ones_v, accm, accc, s0, s1, sa0, sa1):
    cid = lax.axis_index("c")
    sid = lax.axis_index("s")
    wid = sid * NC + cid
    base = wid * RPW
    t0 = sid * NPT
    pltpu.sync_copy(dst_hbm.at[pl.ds(base, RPW)], ds_all)
    pltpu.sync_copy(z32_hbm.at[pl.ds(t0, NPT)], accm.at[pl.ds(t0, NPT)])
    pltpu.sync_copy(z16_hbm.at[pl.ds(t0, NPT)], accc.at[pl.ds(t0, NPT)])
    pltpu.sync_copy(ones_hbm, ones_v)
    plsc.subcore_barrier()

    _scatter_body(m_hbm, dst_hbm, accm, accc, ds_all, m0, m1, ones_v,
                  s0, s1, sa0, sa1, base)

    plsc.subcore_barrier()
    pltpu.sync_copy(accm.at[pl.ds(t0, NPT)], sm_hbm.at[cid, pl.ds(t0, NPT)])
    pltpu.sync_copy(accc.at[pl.ds(t0, NPT)], sc_hbm.at[cid, pl.ds(t0, NPT)])


@functools.partial(
    pl.kernel,
    out_type=jax.ShapeDtypeStruct((NC, N, BIG), jnp.float32),
    mesh=_mesh,
    compiler_params=_sc_params,
    scratch_types=[
        pltpu.VMEM((RPW, RG), jnp.int32),
        pltpu.VMEM((RG, BIG), jnp.float32),
        pltpu.VMEM((RG, BIG), jnp.float32),
        pltpu.VMEM_SHARED((N, BIG), jnp.float32),
        pltpu.SemaphoreType.DMA,
        pltpu.SemaphoreType.DMA,
        pltpu.SemaphoreType.DMA,
        pltpu.SemaphoreType.DMA,
    ],
)
def _sc_scatter(m_hbm, dst_hbm, z32_hbm, sm_hbm, ds_all, m0, m1, accm,
                s0, s1, sa0, sa1):
    cid = lax.axis_index("c")
    sid = lax.axis_index("s")
    wid = sid * NC + cid
    base = wid * RPW
    t0 = sid * NPT
    pltpu.sync_copy(dst_hbm.at[pl.ds(base, RPW)], ds_all)
    pltpu.sync_copy(z32_hbm.at[pl.ds(t0, NPT)], accm.at[pl.ds(t0, NPT)])
    plsc.subcore_barrier()

    _scatter_body(m_hbm, dst_hbm, accm, None, ds_all, m0, m1, None,
                  s0, s1, sa0, sa1, base)

    plsc.subcore_barrier()
    pltpu.sync_copy(accm.at[pl.ds(t0, NPT)], sm_hbm.at[cid, pl.ds(t0, NPT)])


def _tc_prep_body(x_ref, g_ref, bt_ref, w1d_ref, w1b_ref, b1_ref, p_ref, q_ref):
    x = x_ref[...]
    mean = jnp.mean(x, axis=0, keepdims=True)
    xc = x - mean
    var = jnp.mean(xc * xc, axis=0, keepdims=True)
    xn = xc * lax.rsqrt(var + 1e-5) * g_ref[...] + bt_ref[...]
    p_ref[...] = (
        jnp.dot(xn, w1d_ref[...], preferred_element_type=jnp.float32) + b1_ref[...]
    )
    q_ref[...] = jnp.dot(xn, w1b_ref[...], preferred_element_type=jnp.float32)


_tc_prep = pl.pallas_call(
    _tc_prep_body,
    out_shape=(
        jax.ShapeDtypeStruct((N, BIG), jnp.float32),
        jax.ShapeDtypeStruct((N, BIG), jnp.float32),
    ),
)


EPACK = E // 4
BE = 8000


def _tc_mlp_body(h_ref, w_ref, b_ref, m_ref):
    h = jnp.maximum(h_ref[...], 0.0)
    m = jnp.dot(h, w_ref[...], preferred_element_type=jnp.float32) + b_ref[...]
    m_ref[...] = jnp.maximum(m, 0.0)


_tc_mlp = pl.pallas_call(
    _tc_mlp_body,
    grid=(EPACK // BE,),
    in_specs=[
        pl.BlockSpec((BE, D), lambda i: (i, 0)),
        pl.BlockSpec((D, D), lambda i: (0, 0)),
        pl.BlockSpec((1, D), lambda i: (0, 0)),
    ],
    out_specs=pl.BlockSpec((BE, D), lambda i: (i, 0)),
    out_shape=jax.ShapeDtypeStruct((EPACK, D), jnp.float32),
)


def _tc_mid_body(sm_ref, sc_ref, eps_ref, wmu_ref, bmu_ref, wvar_ref, bvar_ref,
                 wd1d_ref, wd1b_ref, bd1_ref,
                 mu_ref, lv_ref, r_ref, s_ref, cnt_ref):
    s = sm_ref[0] + sm_ref[1]
    cnt = sc_ref[0, :, 0:1] + sc_ref[1, :, 0:1]
    henc = s / jnp.maximum(cnt, 1.0)
    mu = jnp.dot(henc, wmu_ref[...], preferred_element_type=jnp.float32) + bmu_ref[...]
    lv = jnp.dot(henc, wvar_ref[...], preferred_element_type=jnp.float32) + bvar_ref[...]
    z = mu + eps_ref[...] * jnp.exp(0.5 * lv)
    z0 = z[:, 0:1]
    z1 = z[:, 1:2]
    r_ref[...] = z0 * wd1d_ref[0:1, :] + z1 * wd1d_ref[1:2, :] + bd1_ref[...]
    s_ref[...] = z0 * wd1b_ref[0:1, :] + z1 * wd1b_ref[1:2, :]
    mu_ref[...] = mu
    lv_ref[...] = lv
    cnt_ref[...] = cnt


_tc_mid = pl.pallas_call(
    _tc_mid_body,
    out_shape=(
        jax.ShapeDtypeStruct((N, HLAT), jnp.float32),
        jax.ShapeDtypeStruct((N, HLAT), jnp.float32),
        jax.ShapeDtypeStruct((N, BIG), jnp.float32),
        jax.ShapeDtypeStruct((N, BIG), jnp.float32),
        jax.ShapeDtypeStruct((N, 1), jnp.float32),
    ),
)


def _tc_out_body(tm_ref, cnt_ref, wd3_ref, bd3_ref, out_ref):
    t = tm_ref[0] + tm_ref[1]
    cnt = cnt_ref[...]
    tmean = t / jnp.maximum(cnt, 1.0)
    mask = jnp.where(cnt > 0, 1.0, 0.0)
    out_ref[...] = (
        jnp.dot(tmean, wd3_ref[...], preferred_element_type=jnp.float32)
        + bd3_ref[...] * mask
    )


_tc_out = pl.pallas_call(
    _tc_out_body,
    out_shape=jax.ShapeDtypeStruct((N, D), jnp.float32),
)


def kernel(x, edge_index, eps, gamma, beta, W1, b1, W2, b2, Wmu, bmu, Wvar, bvar,
           Wd1, bd1, Wd2, bd2, Wd3, bd3):
    src = edge_index[0].reshape(NROWS, RG)
    dst = edge_index[1].reshape(NROWS, RG)
    w1a, w1b = W1[:D], W1[D:]

    p_tab, q_tab = _tc_prep(
        x, gamma.reshape(1, D), beta.reshape(1, D), w1a - w1b, w1b, b1.reshape(1, BIG)
    )

    eye4 = jnp.eye(4, dtype=jnp.float32)
    w2blk = jnp.kron(eye4, W2)
    b2t = jnp.tile(b2, 4).reshape(1, D)

    h1 = _sc_gather(p_tab, q_tab, dst, src)
    m1 = _tc_mlp(h1.reshape(EPACK, D), w2blk, b2t)

    z32 = jnp.zeros((N, BIG), jnp.float32)
    z16 = jnp.zeros((N, 16), jnp.float32)
    ones16 = jnp.ones((RG, 16), jnp.float32)
    sm, sc = _sc_scatter_cnt(m1.reshape(NROWS, RG, BIG), dst, z32, z16, ones16)

    wd1a, wd1b = Wd1[:HLAT], Wd1[HLAT:]
    mu, lv, r_tab, s_tab, cnt = _tc_mid(
        sm, sc, eps, Wmu, bmu.reshape(1, HLAT), Wvar, bvar.reshape(1, HLAT),
        wd1a - wd1b, wd1b, bd1.reshape(1, BIG)
    )

    wd2blk = jnp.kron(eye4, Wd2)
    bd2t = jnp.tile(bd2, 4).reshape(1, D)
    h2 = _sc_gather(r_tab, s_tab, dst, src)
    m2 = _tc_mlp(h2.reshape(EPACK, D), wd2blk, bd2t)
    tm = _sc_scatter(m2.reshape(NROWS, RG, BIG), dst, z32)

    out = _tc_out(tm, cnt, Wd3, bd3.reshape(1, D))
    return (out, mu, lv)


# speedup vs baseline: 13.3172x; 1.0856x over previous
import functools

import jax
import jax.numpy as jnp
from jax import lax
from jax.experimental import pallas as pl
from jax.experimental.pallas import tpu as pltpu
from jax.experimental.pallas import tpu_sc as plsc

N = 10000
E = 320000
D = 128
BIG = 32
HLAT = 2

NC = 2
NS = 16
NW = NC * NS
RG = 125
NROWS = E // RG
RPW = NROWS // NW
NPT = N // NS

_mesh = plsc.VectorSubcoreMesh(
    core_axis_name="c", subcore_axis_name="s", num_cores=NC, num_subcores=NS
)

_sc_params = pltpu.CompilerParams(use_tc_tiling_on_sc=False)


NBUF = 8
GROUP = RPW // NBUF

_gather_scratch = (
    [pltpu.VMEM((RPW, RG), jnp.int32)] * 2
    + [pltpu.VMEM((RG, BIG), jnp.float32)] * (2 * NBUF)
    + [pltpu.SemaphoreType.DMA] * (2 * NBUF)
)


@functools.partial(
    pl.kernel,
    out_type=jax.ShapeDtypeStruct((NROWS, RG, BIG), jnp.float32),
    mesh=_mesh,
    compiler_params=_sc_params,
    scratch_types=_gather_scratch,
)
def _sc_gather(p_hbm, q_hbm, dst_hbm, src_hbm, h_hbm, ds_all, sr_all, *bufs):
    av = bufs[0:NBUF]
    bv = bufs[NBUF:2 * NBUF]
    gsem = bufs[2 * NBUF:3 * NBUF]
    wsem = bufs[3 * NBUF:4 * NBUF]
    wid = lax.axis_index("s") * NC + lax.axis_index("c")
    base = wid * RPW
    pltpu.sync_copy(dst_hbm.at[pl.ds(base, RPW)], ds_all)
    pltpu.sync_copy(src_hbm.at[pl.ds(base, RPW)], sr_all)

    def fire(row, i):
        pltpu.async_copy(p_hbm.at[ds_all.at[row]], av[i], gsem[i])
        pltpu.async_copy(q_hbm.at[sr_all.at[row]], bv[i], gsem[i])

    def drain_gather(i):
        pltpu.make_async_copy(h_hbm.at[0], av[i], gsem[i]).wait()
        pltpu.make_async_copy(h_hbm.at[0], av[i], gsem[i]).wait()

    def drain_write(i):
        pltpu.make_async_copy(av[i], h_hbm.at[0], wsem[i]).wait()

    def add_rows(i):
        a, b = av[i], bv[i]

        @plsc.parallel_loop(0, RG, unroll=8)
        def _(k):
            a[k, 0:16] = a[k, 0:16] + b[k, 0:16]
            a[k, 16:32] = a[k, 16:32] + b[k, 16:32]

    for i in range(NBUF):
        fire(i, i)

    def body(u, carry):
        r0 = NBUF * u
        for i in range(NBUF):
            drain_gather(i)
            add_rows(i)
            pltpu.async_copy(av[i], h_hbm.at[base + r0 + i], wsem[i])

        @pl.when(u < GROUP - 1)
        def _():
            for i in range(NBUF):
                drain_write(i)
                fire(r0 + NBUF + i, i)

        return carry

    lax.fori_loop(0, GROUP, body, 0)
    for i in range(NBUF):
        drain_write(i)


def _scatter_body(m_hbm, dst_hbm, accm, accc, ds_all, m0, m1, ones_v,
                  s0, s1, sa0, sa1, base):
    def drain_load(mv, sem):
        pltpu.make_async_copy(m_hbm.at[0], mv, sem).wait()

    def drain_add(mv, sem):
        pltpu.make_async_copy(mv, accm.at[ds_all.at[0]], sem).wait()

    pltpu.async_copy(m_hbm.at[base], m0, s0)

    def body(t, carry):
        r0 = 2 * t

        @pl.when(t > 0)
        def _():
            drain_add(m1, sa1)

        pltpu.async_copy(m_hbm.at[base + r0 + 1], m1, s1)
        drain_load(m0, s0)
        pltpu.async_copy(m0, accm.at[ds_all.at[r0]], sa0, add=True)
        if accc is not None:
            pltpu.sync_copy(ones_v, accc.at[ds_all.at[r0]], add=True)

        @pl.when(t < RPW // 2 - 1)
        def _():
            drain_add(m0, sa0)
            pltpu.async_copy(m_hbm.at[base + r0 + 2], m0, s0)

        drain_load(m1, s1)
        pltpu.async_copy(m1, accm.at[ds_all.at[r0 + 1]], sa1, add=True)
        if accc is not None:
            pltpu.sync_copy(ones_v, accc.at[ds_all.at[r0 + 1]], add=True)
        return carry

    lax.fori_loop(0, RPW // 2, body, 0)
    drain_add(m0, sa0)
    drain_add(m1, sa1)


@functools.partial(
    pl.kernel,
    out_type=(
        jax.ShapeDtypeStruct((NC, N, BIG), jnp.float32),
        jax.ShapeDtypeStruct((NC, N, 16), jnp.float32),
    ),
    mesh=_mesh,
    compiler_params=_sc_params,
    scratch_types=[
        pltpu.VMEM((RPW, RG), jnp.int32),
        pltpu.VMEM((RG, BIG), jnp.float32),
        pltpu.VMEM((RG, BIG), jnp.float32),
        pltpu.VMEM((RG, 16), jnp.float32),
        pltpu.VMEM_SHARED((N, BIG), jnp.float32),
        pltpu.VMEM_SHARED((N, 16), jnp.float32),
        pltpu.SemaphoreType.DMA,
        pltpu.SemaphoreType.DMA,
        pltpu.SemaphoreType.DMA,
        pltpu.SemaphoreType.DMA,
    ],
)
def _sc_scatter_cnt(m_hbm, dst_hbm, z32_hbm, z16_hbm, ones_hbm, sm_hbm, sc_hbm,
                    ds_all, m0, m1, ones_v, accm, accc, s0, s1, sa0, sa1):
    cid = lax.axis_index("c")
    sid = lax.axis_index("s")
    wid = sid * NC + cid
    base = wid * RPW
    t0 = sid * NPT
    pltpu.sync_copy(dst_hbm.at[pl.ds(base, RPW)], ds_all)
    pltpu.sync_copy(z32_hbm.at[pl.ds(t0, NPT)], accm.at[pl.ds(t0, NPT)])
    pltpu.sync_copy(z16_hbm.at[pl.ds(t0, NPT)], accc.at[pl.ds(t0, NPT)])
    pltpu.sync_copy(ones_hbm, ones_v)
    plsc.subcore_barrier()

    _scatter_body(m_hbm, dst_hbm, accm, accc, ds_all, m0, m1, ones_v,
                  s0, s1, sa0, sa1, base)

    plsc.subcore_barrier()
    pltpu.sync_copy(accm.at[pl.ds(t0, NPT)], sm_hbm.at[cid, pl.ds(t0, NPT)])
    pltpu.sync_copy(accc.at[pl.ds(t0, NPT)], sc_hbm.at[cid, pl.ds(t0, NPT)])


@functools.partial(
    pl.kernel,
    out_type=jax.ShapeDtypeStruct((NC, N, BIG), jnp.float32),
    mesh=_mesh,
    compiler_params=_sc_params,
    scratch_types=[
        pltpu.VMEM((RPW, RG), jnp.int32),
        pltpu.VMEM((RG, BIG), jnp.float32),
        pltpu.VMEM((RG, BIG), jnp.float32),
        pltpu.VMEM_SHARED((N, BIG), jnp.float32),
        pltpu.SemaphoreType.DMA,
        pltpu.SemaphoreType.DMA,
        pltpu.SemaphoreType.DMA,
        pltpu.SemaphoreType.DMA,
    ],
)
def _sc_scatter(m_hbm, dst_hbm, z32_hbm, sm_hbm, ds_all, m0, m1, accm,
                s0, s1, sa0, sa1):
    cid = lax.axis_index("c")
    sid = lax.axis_index("s")
    wid = sid * NC + cid
    base = wid * RPW
    t0 = sid * NPT
    pltpu.sync_copy(dst_hbm.at[pl.ds(base, RPW)], ds_all)
    pltpu.sync_copy(z32_hbm.at[pl.ds(t0, NPT)], accm.at[pl.ds(t0, NPT)])
    plsc.subcore_barrier()

    _scatter_body(m_hbm, dst_hbm, accm, None, ds_all, m0, m1, None,
                  s0, s1, sa0, sa1, base)

    plsc.subcore_barrier()
    pltpu.sync_copy(accm.at[pl.ds(t0, NPT)], sm_hbm.at[cid, pl.ds(t0, NPT)])


def _tc_prep_body(x_ref, g_ref, bt_ref, w1d_ref, w1b_ref, b1_ref, p_ref, q_ref):
    x = x_ref[...]
    mean = jnp.mean(x, axis=0, keepdims=True)
    xc = x - mean
    var = jnp.mean(xc * xc, axis=0, keepdims=True)
    xn = xc * lax.rsqrt(var + 1e-5) * g_ref[...] + bt_ref[...]
    p_ref[...] = (
        jnp.dot(xn, w1d_ref[...], preferred_element_type=jnp.float32) + b1_ref[...]
    )
    q_ref[...] = jnp.dot(xn, w1b_ref[...], preferred_element_type=jnp.float32)


_tc_prep = pl.pallas_call(
    _tc_prep_body,
    out_shape=(
        jax.ShapeDtypeStruct((N, BIG), jnp.float32),
        jax.ShapeDtypeStruct((N, BIG), jnp.float32),
    ),
)


EPACK = E // 4
BE = 8000


def _tc_mlp_body(h_ref, w_ref, b_ref, m_ref):
    h = jnp.maximum(h_ref[...], 0.0)
    m = jnp.dot(h, w_ref[...], preferred_element_type=jnp.float32) + b_ref[...]
    m_ref[...] = jnp.maximum(m, 0.0)


_tc_mlp = pl.pallas_call(
    _tc_mlp_body,
    grid=(EPACK // BE,),
    in_specs=[
        pl.BlockSpec((BE, D), lambda i: (i, 0)),
        pl.BlockSpec((D, D), lambda i: (0, 0)),
        pl.BlockSpec((1, D), lambda i: (0, 0)),
    ],
    out_specs=pl.BlockSpec((BE, D), lambda i: (i, 0)),
    out_shape=jax.ShapeDtypeStruct((EPACK, D), jnp.float32),
)


def _tc_mid_body(sm_ref, sc_ref, eps_ref, wmu_ref, bmu_ref, wvar_ref, bvar_ref,
                 wd1d_ref, wd1b_ref, bd1_ref,
                 mu_ref, lv_ref, r_ref, s_ref, cnt_ref):
    s = sm_ref[0] + sm_ref[1]
    cnt = sc_ref[0, :, 0:1] + sc_ref[1, :, 0:1]
    henc = s / jnp.maximum(cnt, 1.0)
    mu = jnp.dot(henc, wmu_ref[...], preferred_element_type=jnp.float32) + bmu_ref[...]
    lv = jnp.dot(henc, wvar_ref[...], preferred_element_type=jnp.float32) + bvar_ref[...]
    z = mu + eps_ref[...] * jnp.exp(0.5 * lv)
    z0 = z[:, 0:1]
    z1 = z[:, 1:2]
    r_ref[...] = z0 * wd1d_ref[0:1, :] + z1 * wd1d_ref[1:2, :] + bd1_ref[...]
    s_ref[...] = z0 * wd1b_ref[0:1, :] + z1 * wd1b_ref[1:2, :]
    mu_ref[...] = mu
    lv_ref[...] = lv
    cnt_ref[...] = cnt


_tc_mid = pl.pallas_call(
    _tc_mid_body,
    out_shape=(
        jax.ShapeDtypeStruct((N, HLAT), jnp.float32),
        jax.ShapeDtypeStruct((N, HLAT), jnp.float32),
        jax.ShapeDtypeStruct((N, BIG), jnp.float32),
        jax.ShapeDtypeStruct((N, BIG), jnp.float32),
        jax.ShapeDtypeStruct((N, 1), jnp.float32),
    ),
)


def _tc_out_body(tm_ref, cnt_ref, wd3_ref, bd3_ref, out_ref):
    t = tm_ref[0] + tm_ref[1]
    cnt = cnt_ref[...]
    tmean = t / jnp.maximum(cnt, 1.0)
    mask = jnp.where(cnt > 0, 1.0, 0.0)
    out_ref[...] = (
        jnp.dot(tmean, wd3_ref[...], preferred_element_type=jnp.float32)
        + bd3_ref[...] * mask
    )


_tc_out = pl.pallas_call(
    _tc_out_body,
    out_shape=jax.ShapeDtypeStruct((N, D), jnp.float32),
)


def kernel(x, edge_index, eps, gamma, beta, W1, b1, W2, b2, Wmu, bmu, Wvar, bvar,
           Wd1, bd1, Wd2, bd2, Wd3, bd3):
    src = edge_index[0].reshape(NROWS, RG)
    dst = edge_index[1].reshape(NROWS, RG)
    w1a, w1b = W1[:D], W1[D:]

    p_tab, q_tab = _tc_prep(
        x, gamma.reshape(1, D), beta.reshape(1, D), w1a - w1b, w1b, b1.reshape(1, BIG)
    )

    eye4 = jnp.eye(4, dtype=jnp.float32)
    w2blk = jnp.kron(eye4, W2)
    b2t = jnp.tile(b2, 4).reshape(1, D)

    h1 = _sc_gather(p_tab, q_tab, dst, src)
    m1 = _tc_mlp(h1.reshape(EPACK, D), w2blk, b2t)

    z32 = jnp.zeros((N, BIG), jnp.float32)
    z16 = jnp.zeros((N, 16), jnp.float32)
    ones16 = jnp.ones((RG, 16), jnp.float32)
    sm, sc = _sc_scatter_cnt(m1.reshape(NROWS, RG, BIG), dst, z32, z16, ones16)

    wd1a, wd1b = Wd1[:HLAT], Wd1[HLAT:]
    mu, lv, r_tab, s_tab, cnt = _tc_mid(
        sm, sc, eps, Wmu, bmu.reshape(1, HLAT), Wvar, bvar.reshape(1, HLAT),
        wd1a - wd1b, wd1b, bd1.reshape(1, BIG)
    )

    wd2blk = jnp.kron(eye4, Wd2)
    bd2t = jnp.tile(bd2, 4).reshape(1, D)
    h2 = _sc_gather(r_tab, s_tab, dst, src)
    m2 = _tc_mlp(h2.reshape(EPACK, D), wd2blk, bd2t)
    tm = _sc_scatter(m2.reshape(NROWS, RG, BIG), dst, z32)

    out = _tc_out(tm, cnt, Wd3, bd3.reshape(1, D))
    return (out, mu, lv)
